# baseline (speedup 1.0000x reference)
import functools

import jax
import jax.numpy as jnp
from jax import lax
from jax.experimental import pallas as pl
from jax.experimental.pallas import tpu as pltpu

_VMEM_LIMIT = 48 * 1024 * 1024
_BF = jnp.bfloat16


def _lat_kernel(x_ref, w_ref, b_ref, o_ref):
    x = x_ref[0].astype(_BF)
    y = lax.dot_general(x, w_ref[...],
                        dimension_numbers=(((0,), (0,)), ((), ())),
                        preferred_element_type=jnp.float32)
    o_ref[0] = (y + b_ref[...].astype(jnp.float32)).astype(_BF)


def _lat1x1(x, w, b):
    N, Cin, H, W = x.shape
    F = w.shape[1]
    M = H * W
    xf = x.reshape(N, Cin, M)
    return pl.pallas_call(
        _lat_kernel,
        out_shape=jax.ShapeDtypeStruct((N, M, F), _BF),
        grid=(N,),
        in_specs=[
            pl.BlockSpec((1, Cin, M), lambda n: (n, 0, 0)),
            pl.BlockSpec((Cin, F), lambda n: (0, 0)),
            pl.BlockSpec((1, F), lambda n: (0, 0)),
        ],
        out_specs=pl.BlockSpec((1, M, F), lambda n: (n, 0, 0)),
        compiler_params=pltpu.CompilerParams(
            dimension_semantics=("parallel",),
            vmem_limit_bytes=_VMEM_LIMIT),
    )(xf, w, b.reshape(1, F))


def _lat_upadd_kernel(x_ref, w_ref, b_ref, r_ref, o_ref, *, TH2, W):
    F = w_ref.shape[1]
    W2 = r_ref.shape[2]
    x = x_ref[0].astype(_BF)
    y = lax.dot_general(x, w_ref[...],
                        dimension_numbers=(((0,), (0,)), ((), ())),
                        preferred_element_type=jnp.float32)

    r = r_ref[0]
    wf = lax.broadcasted_iota(jnp.int32, (W, W2), 0)
    wc = lax.broadcasted_iota(jnp.int32, (W, W2), 1)
    up = (wc == wf // 2).astype(_BF)
    upb = jnp.broadcast_to(up[None], (TH2, W, W2))
    r_up = lax.dot_general(upb, r,
                           dimension_numbers=(((2,), (1,)), ((0,), (0,))),
                           preferred_element_type=jnp.float32)

    bias = b_ref[...].astype(jnp.float32).reshape(1, 1, 1, F)
    out = y.reshape(TH2, 2, W, F) + bias + r_up[:, None, :, :]
    o_ref[0] = out.reshape(TH2 * 2, W, F).astype(_BF)


def _lat1x1_upadd(x, w, b, r, n_tiles):
    N, Cin, H, W = x.shape
    F = w.shape[1]
    H2, W2 = H // 2, W // 2
    TH = H // n_tiles
    TH2 = TH // 2
    r4 = r.reshape(N, H2, W2, F)
    xf = x.reshape(N, Cin, H * W)
    body = functools.partial(_lat_upadd_kernel, TH2=TH2, W=W)
    return pl.pallas_call(
        body,
        out_shape=jax.ShapeDtypeStruct((N, H, W, F), _BF),
        grid=(N, n_tiles),
        in_specs=[
            pl.BlockSpec((1, Cin, TH * W), lambda n, i: (n, 0, i)),
            pl.BlockSpec((Cin, F), lambda n, i: (0, 0)),
            pl.BlockSpec((1, F), lambda n, i: (0, 0)),
            pl.BlockSpec((1, TH2, W2, F), lambda n, i: (n, i, 0, 0)),
        ],
        out_specs=pl.BlockSpec((1, TH, W, F), lambda n, i: (n, i, 0, 0)),
        compiler_params=pltpu.CompilerParams(
            dimension_semantics=("parallel", "parallel"),
            vmem_limit_bytes=_VMEM_LIMIT),
    )(xf, w, b.reshape(1, F), r4)


def _c3s1_kernel(x_ref, w_ref, b_ref, o_ref, acc_ref, *, TH, W, Cout):
    i = pl.program_id(1)
    bias = b_ref[...].astype(jnp.float32).reshape(1, 1, Cout)
    acc_ref[...] = jnp.broadcast_to(bias, (TH, W, Cout))

    def tap(lhs, k):
        return lax.dot_general(lhs, w_ref[k],
                               dimension_numbers=(((2,), (0,)), ((), ())),
                               preferred_element_type=jnp.float32)

    for dy in range(3):
        rows = x_ref[0, pl.ds(i * TH + dy, TH)]
        acc_ref[...] += tap(rows, 3 * dy + 1)
        acc_ref[:, 1:W] += tap(rows[:, 0:W - 1], 3 * dy + 0)
        acc_ref[:, 0:W - 1] += tap(rows[:, 1:W], 3 * dy + 2)

    o_ref[0] = jnp.transpose(acc_ref[...].reshape(TH * W, Cout))


def _pick_tiles(H, W, cap=6):
    for n in range(cap, 1, -1):
        if H % n == 0 and ((H // n) * W) % 128 == 0:
            return n
    return 1


def _conv3x3_s1(x, w9, b, n_tiles):
    N, H, W, Cin = x.shape
    Cout = w9.shape[-1]
    TH = H // n_tiles
    xp = jnp.pad(x, ((0, 0), (1, 1), (0, 0), (0, 0)))
    body = functools.partial(_c3s1_kernel, TH=TH, W=W, Cout=Cout)
    return pl.pallas_call(
        body,
        out_shape=jax.ShapeDtypeStruct((N, Cout, H * W), jnp.float32),
        grid=(N, n_tiles),
        in_specs=[
            pl.BlockSpec((1, H + 2, W, Cin), lambda n, i: (n, 0, 0, 0)),
            pl.BlockSpec((9, Cin, Cout), lambda n, i: (0, 0, 0)),
            pl.BlockSpec((1, Cout), lambda n, i: (0, 0)),
        ],
        out_specs=pl.BlockSpec((1, Cout, TH * W), lambda n, i: (n, 0, i)),
        scratch_shapes=[pltpu.VMEM((TH, W, Cout), jnp.float32)],
        compiler_params=pltpu.CompilerParams(
            dimension_semantics=("parallel", "arbitrary"),
            vmem_limit_bytes=_VMEM_LIMIT),
    )(xp, w9, b.reshape(1, Cout))


def _c3s2_kernel(x_ref, w_ref, b_ref, o_ref, *, H, W, Ho, Wo, F, apply_relu):
    x = x_ref[0].astype(_BF)
    if apply_relu:
        x = jnp.maximum(x, jnp.zeros_like(x))
    M = H * W
    Mo = Ho * Wo

    p_r = lax.broadcasted_iota(jnp.int32, (M, Mo), 0) // W
    p_c = lax.broadcasted_iota(jnp.int32, (M, Mo), 0) % W
    q_i = lax.broadcasted_iota(jnp.int32, (M, Mo), 1) // Wo
    q_j = lax.broadcasted_iota(jnp.int32, (M, Mo), 1) % Wo

    acc = jnp.broadcast_to(b_ref[...].astype(jnp.float32).reshape(F, 1),
                           (F, Mo))
    for dy in range(3):
        for dx in range(3):
            sel = jnp.logical_and(p_r == 2 * q_i + dy - 1,
                                  p_c == 2 * q_j + dx - 1).astype(_BF)
            t1 = jnp.dot(x, sel, preferred_element_type=jnp.float32)
            acc = acc + lax.dot_general(
                w_ref[3 * dy + dx], t1.astype(_BF),
                dimension_numbers=(((0,), (0,)), ((), ())),
                preferred_element_type=jnp.float32)
    o_ref[0] = acc


def _conv3x3_s2(x, w9, b, H, W, apply_relu=False):
    N, Cin, _ = x.shape
    Cout = w9.shape[-1]
    Ho = (H - 1) // 2 + 1
    Wo = (W - 1) // 2 + 1
    body = functools.partial(_c3s2_kernel, H=H, W=W, Ho=Ho, Wo=Wo, F=Cout,
                             apply_relu=apply_relu)
    return pl.pallas_call(
        body,
        out_shape=jax.ShapeDtypeStruct((N, Cout, Ho * Wo), jnp.float32),
        grid=(N,),
        in_specs=[
            pl.BlockSpec((1, Cin, H * W), lambda n: (n, 0, 0)),
            pl.BlockSpec((9, Cin, Cout), lambda n: (0, 0, 0)),
            pl.BlockSpec((1, Cout), lambda n: (0, 0)),
        ],
        out_specs=pl.BlockSpec((1, Cout, Ho * Wo), lambda n: (n, 0, 0)),
        compiler_params=pltpu.CompilerParams(
            dimension_semantics=("parallel",),
            vmem_limit_bytes=_VMEM_LIMIT),
    )(x, w9, b.reshape(1, Cout))


def kernel(C3, C4, C5, P5_1_w, P5_1_b, P5_2_w, P5_2_b, P4_1_w, P4_1_b,
           P4_2_w, P4_2_b, P3_1_w, P3_1_b, P3_2_w, P3_2_b, P6_w, P6_b,
           P7_2_w, P7_2_b):
    N = C3.shape[0]
    F = P5_1_w.shape[1]
    H5, W5 = C5.shape[2], C5.shape[3]
    H4, W4 = C4.shape[2], C4.shape[3]
    H3, W3 = C3.shape[2], C3.shape[3]

    p5_lat = _lat1x1(C5, P5_1_w.astype(_BF), P5_1_b)
    p5 = _conv3x3_s1(p5_lat.reshape(N, H5, W5, F),
                     P5_2_w.astype(_BF), P5_2_b, 1)

    p4_lat = _lat1x1_upadd(C4, P4_1_w.astype(_BF), P4_1_b, p5_lat, 1)
    p4 = _conv3x3_s1(p4_lat, P4_2_w.astype(_BF), P4_2_b,
                     _pick_tiles(H4, W4))

    p3_lat = _lat1x1_upadd(C3, P3_1_w.astype(_BF), P3_1_b,
                           p4_lat.reshape(N, H4 * W4, F),
                           2 if H3 % 4 == 0 else 1)
    p3 = _conv3x3_s1(p3_lat, P3_2_w.astype(_BF), P3_2_b,
                     _pick_tiles(H3, W3))

    c5f = C5.reshape(N, C5.shape[1], H5 * W5)
    p6 = _conv3x3_s2(c5f, P6_w.astype(_BF), P6_b, H5, W5)
    H6, W6 = (H5 - 1) // 2 + 1, (W5 - 1) // 2 + 1
    p7 = _conv3x3_s2(p6, P7_2_w.astype(_BF), P7_2_b, H6, W6,
                     apply_relu=True)
    H7, W7 = (H6 - 1) // 2 + 1, (W6 - 1) // 2 + 1

    return [p3.reshape(N, F, H3, W3), p4.reshape(N, F, H4, W4),
            p5.reshape(N, F, H5, W5), p6.reshape(N, F, H6, W6),
            p7.reshape(N, F, H7, W7)]


# speedup vs baseline: 1.2705x; 1.2705x over previous
import functools

import jax
import jax.numpy as jnp
from jax import lax
from jax.experimental import pallas as pl
from jax.experimental.pallas import tpu as pltpu

_VMEM_LIMIT = 58 * 1024 * 1024
_BF = jnp.bfloat16


def _bias_f32(b_ref):
    return b_ref[...].astype(jnp.float32).reshape(1, 1, -1)


def _lateral_into(x_ref, w_ref, b_ref, lat_ref, *, H, chunk):
    bias = _bias_f32(b_ref)
    for c0 in range(0, H, chunk):
        y = lax.dot_general(x_ref[0, c0:c0 + chunk], w_ref[...],
                            dimension_numbers=(((2,), (0,)), ((), ())),
                            preferred_element_type=jnp.float32)
        lat_ref[0, c0:c0 + chunk] = (y + bias).astype(_BF)


def _upsample_add_into(x_ref, w_ref, b_ref, r_ref, lat_ref, *, H, W, chunk):
    bias = _bias_f32(b_ref)
    F = w_ref.shape[1]
    W2 = W // 2
    wf = lax.broadcasted_iota(jnp.int32, (W, W2), 0)
    wc = lax.broadcasted_iota(jnp.int32, (W, W2), 1)
    up = jnp.broadcast_to(((wc == wf // 2).astype(_BF))[None],
                          (chunk // 2, W, W2))
    for c0 in range(0, H, chunk):
        y = lax.dot_general(x_ref[0, c0:c0 + chunk], w_ref[...],
                            dimension_numbers=(((2,), (0,)), ((), ())),
                            preferred_element_type=jnp.float32)
        r = r_ref[0, c0 // 2:(c0 + chunk) // 2]
        r_up = lax.dot_general(up, r,
                               dimension_numbers=(((2,), (1,)), ((0,), (0,))),
                               preferred_element_type=jnp.float32)
        out = (y.reshape(chunk // 2, 2, W, F) + bias[None]
               + r_up[:, None, :, :])
        lat_ref[0, c0:c0 + chunk] = out.reshape(chunk, W, F).astype(_BF)


def _conv3x3_into(lat_ref, w_ref, b_ref, o_ref, acc_ref, *, H, W, chunk):
    bias = _bias_f32(b_ref)
    F = w_ref.shape[2]

    def tap(lhs, k):
        return lax.dot_general(lhs, w_ref[k],
                               dimension_numbers=(((2,), (0,)), ((), ())),
                               preferred_element_type=jnp.float32)

    for c0 in range(0, H, chunk):
        acc_ref[...] = jnp.broadcast_to(bias, (chunk, W, F))
        for dy in range(3):
            d_lo = max(0, 1 - dy - c0)
            d_hi = min(chunk, H - c0 - dy + 1)
            rows = lat_ref[0, c0 + d_lo + dy - 1:c0 + d_hi + dy - 1]
            acc_ref[d_lo:d_hi] += tap(rows, 3 * dy + 1)
            acc_ref[d_lo:d_hi, 1:W] += tap(rows[:, 0:W - 1], 3 * dy + 0)
            acc_ref[d_lo:d_hi, 0:W - 1] += tap(rows[:, 1:W], 3 * dy + 2)
        o_ref[0, c0:c0 + chunk] = acc_ref[...].astype(_BF)


def _p54_kernel(c4_ref, c5_ref,
                w5l_ref, b5l_ref, w4l_ref, b4l_ref,
                w5s_ref, b5s_ref, w4s_ref, b4s_ref,
                o5_ref, o4_ref, olat4_ref,
                lat5_ref, acc5_ref, acc4_ref, *, H5, W5, H4, W4):
    _lateral_into(c5_ref, w5l_ref, b5l_ref, lat5_ref, H=H5, chunk=H5)
    _conv3x3_into(lat5_ref, w5s_ref, b5s_ref, o5_ref, acc5_ref,
                  H=H5, W=W5, chunk=H5)
    _upsample_add_into(c4_ref, w4l_ref, b4l_ref, lat5_ref, olat4_ref,
                       H=H4, W=W4, chunk=H4 // 2)
    _conv3x3_into(olat4_ref, w4s_ref, b4s_ref, o4_ref, acc4_ref,
                  H=H4, W=W4, chunk=H4 // 2)


def _p3_kernel(c3_ref, lat4_ref, w3l_ref, b3l_ref, w3s_ref, b3s_ref,
               o3_ref, lat3_ref, acc3_ref, *, H3, W3):
    _upsample_add_into(c3_ref, w3l_ref, b3l_ref, lat4_ref, lat3_ref,
                       H=H3, W=W3, chunk=H3 // 4)
    _conv3x3_into(lat3_ref, w3s_ref, b3s_ref, o3_ref, acc3_ref,
                  H=H3, W=W3, chunk=H3 // 4)


def _s2_conv_from(x, w_ref, b_ref, acc_ref, *, H, W, F):
    H2 = H // 2
    Wo = (W - 1) // 2 + 1
    Cin = x.shape[-1]
    x5 = x.reshape(H2, 2, W, Cin)
    acc_ref[...] = jnp.broadcast_to(_bias_f32(b_ref), (H2, W, F))

    def tap(lhs, k):
        return lax.dot_general(lhs, w_ref[k],
                               dimension_numbers=(((2,), (0,)), ((), ())),
                               preferred_element_type=jnp.float32)

    def cols(rows, ky, ro, nr):
        acc_ref[ro:ro + nr] += tap(rows, 3 * ky + 1)
        acc_ref[ro:ro + nr, 1:W] += tap(rows[:, 0:W - 1], 3 * ky + 0)
        acc_ref[ro:ro + nr, 0:W - 1] += tap(rows[:, 1:W], 3 * ky + 2)

    cols(x5[:, 0], 1, 0, H2)
    cols(x5[:, 1], 2, 0, H2)
    if H2 > 1:
        cols(x5[0:H2 - 1, 1], 0, 1, H2 - 1)

    wo = lax.broadcasted_iota(jnp.int32, (Wo, W), 0)
    wi = lax.broadcasted_iota(jnp.int32, (Wo, W), 1)
    sel = jnp.broadcast_to(((wi == 2 * wo).astype(jnp.float32))[None],
                           (H2, Wo, W))
    return lax.dot_general(sel, acc_ref[...],
                           dimension_numbers=(((2,), (1,)), ((0,), (0,))),
                           preferred_element_type=jnp.float32)


def _p67_kernel(c5_ref, w6_ref, b6_ref, w7_ref, b7_ref, o6_ref, o7_ref,
                acc6_ref, acc7_ref, *, H5, W5, F):
    p6 = _s2_conv_from(c5_ref[0], w6_ref, b6_ref, acc6_ref,
                       H=H5, W=W5, F=F)
    o6_ref[0] = p6.astype(_BF)
    p6r = jnp.maximum(p6, 0.0).astype(_BF)
    H6 = H5 // 2
    W6 = (W5 - 1) // 2 + 1
    p7 = _s2_conv_from(p6r, w7_ref, b7_ref, acc7_ref, H=H6, W=W6, F=F)
    o7_ref[0] = p7.astype(_BF)


def kernel(C3, C4, C5, P5_1_w, P5_1_b, P5_2_w, P5_2_b, P4_1_w, P4_1_b,
           P4_2_w, P4_2_b, P3_1_w, P3_1_b, P3_2_w, P3_2_b, P6_w, P6_b,
           P7_2_w, P7_2_b):
    N, C3c, H3, W3 = C3.shape
    _, C4c, H4, W4 = C4.shape
    _, C5c, H5, W5 = C5.shape
    F = P5_1_w.shape[1]

    to_nhwc = lambda t: jnp.transpose(t.astype(_BF), (0, 2, 3, 1))
    c3 = to_nhwc(C3)
    c4 = to_nhwc(C4)
    c5 = to_nhwc(C5)
    bf = lambda w: w.astype(_BF)
    b2 = lambda b: b.reshape(1, F)

    full = lambda *shape: pl.BlockSpec(shape, lambda n: (0,) * len(shape))
    img = lambda H, W, C: pl.BlockSpec((1, H, W, C), lambda n: (n, 0, 0, 0))
    cp = pltpu.CompilerParams(dimension_semantics=("parallel",),
                              vmem_limit_bytes=_VMEM_LIMIT)

    body_a = functools.partial(_p54_kernel, H5=H5, W5=W5, H4=H4, W4=W4)
    o5, o4, lat4 = pl.pallas_call(
        body_a,
        out_shape=[jax.ShapeDtypeStruct((N, H5, W5, F), _BF),
                   jax.ShapeDtypeStruct((N, H4, W4, F), _BF),
                   jax.ShapeDtypeStruct((N, H4, W4, F), _BF)],
        grid=(N,),
        in_specs=[
            img(H4, W4, C4c), img(H5, W5, C5c),
            full(C5c, F), full(1, F), full(C4c, F), full(1, F),
            full(9, F, F), full(1, F), full(9, F, F), full(1, F),
        ],
        out_specs=[img(H5, W5, F), img(H4, W4, F), img(H4, W4, F)],
        scratch_shapes=[
            pltpu.VMEM((1, H5, W5, F), _BF),
            pltpu.VMEM((H5, W5, F), jnp.float32),
            pltpu.VMEM((H4 // 2, W4, F), jnp.float32),
        ],
        compiler_params=cp,
    )(c4, c5, bf(P5_1_w), b2(P5_1_b), bf(P4_1_w), b2(P4_1_b),
      bf(P5_2_w), b2(P5_2_b), bf(P4_2_w), b2(P4_2_b))

    body_b = functools.partial(_p3_kernel, H3=H3, W3=W3)
    o3 = pl.pallas_call(
        body_b,
        out_shape=jax.ShapeDtypeStruct((N, H3, W3, F), _BF),
        grid=(N,),
        in_specs=[
            img(H3, W3, C3c), img(H4, W4, F),
            full(C3c, F), full(1, F), full(9, F, F), full(1, F),
        ],
        out_specs=img(H3, W3, F),
        scratch_shapes=[
            pltpu.VMEM((1, H3, W3, F), _BF),
            pltpu.VMEM((H3 // 4, W3, F), jnp.float32),
        ],
        compiler_params=cp,
    )(c3, lat4, bf(P3_1_w), b2(P3_1_b), bf(P3_2_w), b2(P3_2_b))

    H6, W6 = H5 // 2, (W5 - 1) // 2 + 1
    H7, W7 = H6 // 2, (W6 - 1) // 2 + 1
    body_c = functools.partial(_p67_kernel, H5=H5, W5=W5, F=F)
    o6, o7 = pl.pallas_call(
        body_c,
        out_shape=[jax.ShapeDtypeStruct((N, H6, W6, F), _BF),
                   jax.ShapeDtypeStruct((N, H7, W7, F), _BF)],
        grid=(N,),
        in_specs=[
            img(H5, W5, C5c),
            full(9, C5c, F), full(1, F), full(9, F, F), full(1, F),
        ],
        out_specs=[img(H6, W6, F), img(H7, W7, F)],
        scratch_shapes=[
            pltpu.VMEM((H5 // 2, W5, F), jnp.float32),
            pltpu.VMEM((H6 // 2, W6, F), jnp.float32),
        ],
        compiler_params=cp,
    )(c5, bf(P6_w), b2(P6_b), bf(P7_2_w), b2(P7_2_b))

    fin = lambda t: jnp.transpose(t, (0, 3, 1, 2)).astype(jnp.float32)
    return [fin(o3), fin(o4), fin(o5), fin(o6), fin(o7)]
